# speedup vs baseline: 8.6125x; 8.6125x over previous
import functools

import jax
import jax.numpy as jnp
from jax import lax
from jax.experimental import pallas as pl
from jax.experimental.pallas import tpu as pltpu
from jax.experimental.pallas import tpu_sc as plsc

_D = 128
_NC = 2
_NS = 16
_NW = _NC * _NS
_LANES = 16


def _linear_body(x_ref, w_ref, b_ref, z_ref):
    z_ref[...] = (
        lax.dot_general(
            x_ref[...], w_ref[...], (((1,), (0,)), ((), ())),
            precision=lax.Precision.HIGHEST,
            preferred_element_type=jnp.float32,
        )
        + b_ref[...]
    )


def _node_linear(x, w_t, b_row, row_block):
    n, d = x.shape
    grid = n // row_block
    return pl.pallas_call(
        _linear_body,
        grid=(grid,),
        in_specs=[
            pl.BlockSpec((row_block, d), lambda i: (i, 0)),
            pl.BlockSpec((d, d), lambda i: (0, 0)),
            pl.BlockSpec((1, d), lambda i: (0, 0)),
        ],
        out_specs=pl.BlockSpec((row_block, d), lambda i: (i, 0)),
        out_shape=jax.ShapeDtypeStruct((n, d), jnp.float32),
    )(x, w_t, b_row)


def _make_edge_gather(n_edges, d, chunk):
    epw = n_edges // _NW
    n_chunks = epw // chunk
    mesh = plsc.VectorSubcoreMesh(core_axis_name="c", subcore_axis_name="s")

    @functools.partial(
        pl.kernel,
        mesh=mesh,
        out_type=jax.ShapeDtypeStruct((n_edges, d), jnp.float32),
        scratch_types=[
            pltpu.VMEM((chunk,), jnp.int32),
            pltpu.VMEM((chunk,), jnp.int32),
            pltpu.VMEM((chunk, d), jnp.float32),
            pltpu.VMEM((chunk, d), jnp.float32),
            pltpu.SemaphoreType.DMA,
            pltpu.SemaphoreType.DMA,
        ],
    )
    def edge_gather(z_hbm, idx0_hbm, idx1_hbm, out_hbm,
                    idx0_v, idx1_v, rows0_v, rows1_v, sem0, sem1):
        wid = lax.axis_index("s") * _NC + lax.axis_index("c")
        base0 = pl.multiple_of(wid * epw, 8)

        def chunk_body(k, carry):
            base = pl.multiple_of(base0 + k * chunk, 8)
            pltpu.sync_copy(idx0_hbm.at[pl.ds(base, chunk)], idx0_v)
            pltpu.sync_copy(idx1_hbm.at[pl.ds(base, chunk)], idx1_v)
            cp0 = pltpu.async_copy(z_hbm.at[idx0_v], rows0_v, sem0)
            cp1 = pltpu.async_copy(z_hbm.at[idx1_v], rows1_v, sem1)
            cp0.wait()
            cp1.wait()

            def add_body(i, c):
                for j in range(d // _LANES):
                    sl = pl.ds(j * _LANES, _LANES)
                    plsc.addupdate(rows0_v.at[i, sl], rows1_v[i, sl])
                return c

            lax.fori_loop(0, chunk, add_body, 0, unroll=2)
            pltpu.sync_copy(rows0_v, out_hbm.at[pl.ds(base, chunk)])
            return carry

        lax.fori_loop(0, n_chunks, chunk_body, 0)

    return edge_gather


def kernel(x, edges, W, b):
    n, d = x.shape
    n_edges = edges.shape[0]
    e32 = edges.astype(jnp.int32)
    idx0 = e32[:, 0]
    idx1 = e32[:, 1]
    w_t = (0.5 * W).T.astype(jnp.float32)
    b_row = (0.5 * b).reshape(1, d).astype(jnp.float32)
    z = _node_linear(x, w_t, b_row, row_block=1000)
    edge_gather = _make_edge_gather(n_edges, d, chunk=80)
    return edge_gather(z, idx0, idx1)


# speedup vs baseline: 18.1946x; 2.1126x over previous
import functools

import jax
import jax.numpy as jnp
from jax import lax
from jax.experimental import pallas as pl
from jax.experimental.pallas import tpu as pltpu
from jax.experimental.pallas import tpu_sc as plsc

_D = 128
_NC = 2
_NS = 16
_NW = _NC * _NS
_LANES = 16
_NBUF = 5
_FLIGHT = 3


def _linear_body(x_ref, w_ref, b_ref, z_ref):
    z_ref[...] = (
        lax.dot_general(
            x_ref[...], w_ref[...], (((1,), (0,)), ((), ())),
            precision=lax.Precision.HIGHEST,
            preferred_element_type=jnp.float32,
        )
        + b_ref[...]
    )


def _node_linear(x, w_t, b_row, row_block):
    n, d = x.shape
    grid = n // row_block
    return pl.pallas_call(
        _linear_body,
        grid=(grid,),
        in_specs=[
            pl.BlockSpec((row_block, d), lambda i: (i, 0)),
            pl.BlockSpec((d, d), lambda i: (0, 0)),
            pl.BlockSpec((1, d), lambda i: (0, 0)),
        ],
        out_specs=pl.BlockSpec((row_block, d), lambda i: (i, 0)),
        out_shape=jax.ShapeDtypeStruct((n, d), jnp.float32),
    )(x, w_t, b_row)


def _make_edge_gather(n_edges, d, chunk):
    epw = n_edges // _NW
    n_chunks = epw // chunk
    n_groups = n_chunks // _NBUF
    assert n_chunks % _NBUF == 0 and n_groups >= 2
    mesh = plsc.VectorSubcoreMesh(core_axis_name="c", subcore_axis_name="s")

    scratch = (
        [pltpu.VMEM((epw,), jnp.int32)] * 2
        + [pltpu.VMEM((chunk, d), jnp.float32)] * (2 * _NBUF)
        + [pltpu.SemaphoreType.DMA] * (3 * _NBUF)
    )

    @functools.partial(
        pl.kernel,
        mesh=mesh,
        out_type=jax.ShapeDtypeStruct((n_edges, d), jnp.float32),
        scratch_types=scratch,
    )
    def edge_gather(z_hbm, idx0_hbm, idx1_hbm, out_hbm, *s):
        idx0_all, idx1_all = s[0], s[1]
        rows0 = s[2:2 + _NBUF]
        rows1 = s[2 + _NBUF:2 + 2 * _NBUF]
        g0sem = s[2 + 2 * _NBUF:2 + 3 * _NBUF]
        g1sem = s[2 + 3 * _NBUF:2 + 4 * _NBUF]
        osem = s[2 + 4 * _NBUF:2 + 5 * _NBUF]

        wid = lax.axis_index("s") * _NC + lax.axis_index("c")
        base0 = pl.multiple_of(wid * epw, 8)
        pltpu.sync_copy(idx0_hbm.at[pl.ds(base0, epw)], idx0_all)
        pltpu.sync_copy(idx1_hbm.at[pl.ds(base0, epw)], idx1_all)

        def gather_cps(b, k):
            off = pl.multiple_of(k * chunk, 8)
            cp0 = pltpu.make_async_copy(
                z_hbm.at[idx0_all.at[pl.ds(off, chunk)]], rows0[b], g0sem[b])
            cp1 = pltpu.make_async_copy(
                z_hbm.at[idx1_all.at[pl.ds(off, chunk)]], rows1[b], g1sem[b])
            return cp0, cp1

        def store_cp(b, k):
            off = pl.multiple_of(base0 + k * chunk, 8)
            return pltpu.make_async_copy(
                rows0[b], out_hbm.at[pl.ds(off, chunk)], osem[b])

        def fire_gathers(b, k):
            cp0, cp1 = gather_cps(b, k)
            cp0.start()
            cp1.start()

        def do_chunk(k, b, refire, wait_prev_store):
            rb = (b + _FLIGHT) % _NBUF
            if refire:
                if wait_prev_store:
                    store_cp(rb, k - (_NBUF - _FLIGHT)).wait()
                fire_gathers(rb, k + _FLIGHT)
            cp0, cp1 = gather_cps(b, k)
            cp0.wait()
            cp1.wait()

            def add_body(i, c):
                for j in range(d // _LANES):
                    sl = pl.ds(j * _LANES, _LANES)
                    plsc.addupdate(rows0[b].at[i, sl], rows1[b][i, sl])
                return c

            lax.fori_loop(0, chunk, add_body, 0, unroll=2)
            store_cp(b, k).start()

        for b in range(_FLIGHT):
            fire_gathers(b, b)
        for b in range(_NBUF):
            do_chunk(b, b, refire=True,
                     wait_prev_store=(b >= _NBUF - _FLIGHT))
        def group_body(g, carry):
            k0 = g * _NBUF
            for b in range(_NBUF):
                do_chunk(k0 + b, b, refire=True, wait_prev_store=True)
            return carry

        lax.fori_loop(1, n_groups - 1, group_body, 0)
        k0 = (n_groups - 1) * _NBUF
        for b in range(_NBUF):
            do_chunk(k0 + b, b, refire=(b < _NBUF - _FLIGHT),
                     wait_prev_store=True)
        for b in range(_NBUF):
            store_cp(b, k0 + b).wait()

    return edge_gather


def kernel(x, edges, W, b):
    n, d = x.shape
    n_edges = edges.shape[0]
    e32 = edges.astype(jnp.int32)
    idx0 = e32[:, 0]
    idx1 = e32[:, 1]
    w_t = (0.5 * W).T.astype(jnp.float32)
    b_row = (0.5 * b).reshape(1, d).astype(jnp.float32)
    z = _node_linear(x, w_t, b_row, row_block=1000)
    edge_gather = _make_edge_gather(n_edges, d, chunk=80)
    return edge_gather(z, idx0, idx1)
